# Initial kernel scaffold; baseline (speedup 1.0000x reference)
import jax
import jax.numpy as jnp
from jax.experimental import pallas as pl


def kernel(x, a, i, W1a, W1b, b1, Wp, bp, W2a, W2b, b2, Wd, bd):
    raise NotImplementedError("write your pallas kernel here")


# speedup vs baseline: 1.2766x; 1.2766x over previous
import functools

import jax
import jax.numpy as jnp
from jax.experimental import pallas as pl
from jax.experimental.pallas import tpu as pltpu


def _body(A_ref, X_ref, W1a_ref, W1b_ref, b1_ref, Wp_ref, bp_ref,
          W2a_ref, W2b_ref, b2_ref, Wd_ref, bd_ref,
          out_ref, P_ref, S_ref, xp_ref, ap_ref, *, BN, NB, K):
    p = pl.program_id(0)
    b = pl.program_id(1)

    @pl.when(jnp.logical_and(p == 0, b == 0))
    def _init():
        P_ref[...] = jnp.dot(X_ref[...], W1a_ref[...],
                             preferred_element_type=jnp.float32)
        xp_ref[...] = jnp.zeros_like(xp_ref)
        ap_ref[...] = jnp.zeros_like(ap_ref)

    @pl.when(p == 0)
    def _phase0():
        A_b = A_ref[...]
        X_b = X_ref[pl.ds(b * BN, BN), :]
        h = jnp.dot(A_b, P_ref[...], preferred_element_type=jnp.float32)
        h = h + jnp.dot(X_b, W1b_ref[...],
                        preferred_element_type=jnp.float32) + b1_ref[...]
        h = jnp.maximum(h, 0.0)
        logits = jnp.dot(h, Wp_ref[...],
                         preferred_element_type=jnp.float32) + bp_ref[...]
        m = jnp.max(logits, axis=-1, keepdims=True)
        e = jnp.exp(logits - m)
        S_b = e / jnp.sum(e, axis=-1, keepdims=True)
        S_ref[pl.ds(b * BN, BN), :] = S_b
        xp_ref[...] += jax.lax.dot_general(
            S_b, h, (((0,), (0,)), ((), ())),
            preferred_element_type=jnp.float32)

    @pl.when(p == 1)
    def _phase1():
        A_b = A_ref[...]
        AS = jnp.dot(A_b, S_ref[...], preferred_element_type=jnp.float32)
        S_b = S_ref[pl.ds(b * BN, BN), :]
        ap_ref[...] += jax.lax.dot_general(
            S_b, AS, (((0,), (0,)), ((), ())),
            preferred_element_type=jnp.float32)

    @pl.when(jnp.logical_and(p == 1, b == NB - 1))
    def _final():
        ap = ap_ref[...]
        r = jax.lax.broadcasted_iota(jnp.int32, (K, K), 0)
        c = jax.lax.broadcasted_iota(jnp.int32, (K, K), 1)
        ap = jnp.where(r == c, 0.0, ap)
        d = jnp.sum(ap, axis=1, keepdims=True)
        dinv = jax.lax.rsqrt(d + 1e-9)
        xp = xp_ref[...]
        u = jnp.dot(xp, W2a_ref[...], preferred_element_type=jnp.float32)
        v = jnp.dot(ap, u * dinv, preferred_element_type=jnp.float32) * dinv
        h2 = v + jnp.dot(xp, W2b_ref[...],
                         preferred_element_type=jnp.float32) + b2_ref[...]
        h2 = jnp.maximum(h2, 0.0)
        out_ref[...] = jnp.dot(h2, Wd_ref[...],
                               preferred_element_type=jnp.float32) + bd_ref[...]


def kernel(x, a, i, W1a, W1b, b1, Wp, bp, W2a, W2b, b2, Wd, bd):
    N, F = x.shape
    H = W1a.shape[1]
    K = Wp.shape[1]
    BN = 256
    NB = N // BN
    body = functools.partial(_body, BN=BN, NB=NB, K=K)
    full = lambda p, b: (0, 0)
    out = pl.pallas_call(
        body,
        grid=(2, NB),
        in_specs=[
            pl.BlockSpec((BN, N), lambda p, b: (b, 0)),
            pl.BlockSpec((N, F), full),
            pl.BlockSpec((F, H), full),
            pl.BlockSpec((F, H), full),
            pl.BlockSpec((1, H), full),
            pl.BlockSpec((H, K), full),
            pl.BlockSpec((1, K), full),
            pl.BlockSpec((H, H), full),
            pl.BlockSpec((H, H), full),
            pl.BlockSpec((1, H), full),
            pl.BlockSpec((H, 1), full),
            pl.BlockSpec((1, 1), full),
        ],
        out_specs=pl.BlockSpec((K, 1), full),
        out_shape=jax.ShapeDtypeStruct((K, 1), jnp.float32),
        scratch_shapes=[
            pltpu.VMEM((N, H), jnp.float32),
            pltpu.VMEM((N, K), jnp.float32),
            pltpu.VMEM((K, H), jnp.float32),
            pltpu.VMEM((K, K), jnp.float32),
        ],
    )(a, x, W1a, W1b, b1.reshape(1, H), Wp, bp.reshape(1, K),
      W2a, W2b, b2.reshape(1, H), Wd, bd.reshape(1, 1))
    return out


# speedup vs baseline: 1.2767x; 1.0000x over previous
import functools

import jax
import jax.numpy as jnp
from jax.experimental import pallas as pl
from jax.experimental.pallas import tpu as pltpu


def _body(A_ref, X_ref, W1a_ref, W1b_ref, b1_ref, Wp_ref, bp_ref,
          W2a_ref, W2b_ref, b2_ref, Wd_ref, bd_ref,
          out_ref, P_ref, Avm_ref, S_ref, xp_ref, ap_ref, *, BN, NB, K):
    p = pl.program_id(0)
    b = pl.program_id(1)

    @pl.when(jnp.logical_and(p == 0, b == 0))
    def _init():
        P_ref[...] = jnp.dot(X_ref[...], W1a_ref[...],
                             preferred_element_type=jnp.float32)
        xp_ref[...] = jnp.zeros_like(xp_ref)
        ap_ref[...] = jnp.zeros_like(ap_ref)

    @pl.when(p == 0)
    def _phase0():
        A_b = A_ref[...]
        Avm_ref[pl.ds(b * BN, BN), :] = A_b.astype(jnp.bfloat16)
        X_b = X_ref[pl.ds(b * BN, BN), :]
        h = jnp.dot(A_b, P_ref[...], preferred_element_type=jnp.float32)
        h = h + jnp.dot(X_b, W1b_ref[...],
                        preferred_element_type=jnp.float32) + b1_ref[...]
        h = jnp.maximum(h, 0.0)
        logits = jnp.dot(h, Wp_ref[...],
                         preferred_element_type=jnp.float32) + bp_ref[...]
        m = jnp.max(logits, axis=-1, keepdims=True)
        e = jnp.exp(logits - m)
        S_b = e / jnp.sum(e, axis=-1, keepdims=True)
        S_ref[pl.ds(b * BN, BN), :] = S_b.astype(jnp.bfloat16)
        xp_ref[...] += jax.lax.dot_general(
            S_b, h, (((0,), (0,)), ((), ())),
            preferred_element_type=jnp.float32)

    @pl.when(p == 1)
    def _phase1():
        A_b = Avm_ref[pl.ds(b * BN, BN), :]
        AS = jnp.dot(A_b, S_ref[...], preferred_element_type=jnp.float32)
        S_b = S_ref[pl.ds(b * BN, BN), :]
        ap_ref[...] += jax.lax.dot_general(
            S_b, AS.astype(jnp.bfloat16), (((0,), (0,)), ((), ())),
            preferred_element_type=jnp.float32)

    @pl.when(jnp.logical_and(p == 1, b == NB - 1))
    def _final():
        ap = ap_ref[...]
        r = jax.lax.broadcasted_iota(jnp.int32, (K, K), 0)
        c = jax.lax.broadcasted_iota(jnp.int32, (K, K), 1)
        ap = jnp.where(r == c, 0.0, ap)
        d = jnp.sum(ap, axis=1, keepdims=True)
        dinv = jax.lax.rsqrt(d + 1e-9)
        xp = xp_ref[...]
        u = jnp.dot(xp, W2a_ref[...], preferred_element_type=jnp.float32)
        v = jnp.dot(ap, u * dinv, preferred_element_type=jnp.float32) * dinv
        h2 = v + jnp.dot(xp, W2b_ref[...],
                         preferred_element_type=jnp.float32) + b2_ref[...]
        h2 = jnp.maximum(h2, 0.0)
        out_ref[...] = jnp.dot(h2, Wd_ref[...],
                               preferred_element_type=jnp.float32) + bd_ref[...]


def kernel(x, a, i, W1a, W1b, b1, Wp, bp, W2a, W2b, b2, Wd, bd):
    N, F = x.shape
    H = W1a.shape[1]
    K = Wp.shape[1]
    BN = 256
    NB = N // BN
    body = functools.partial(_body, BN=BN, NB=NB, K=K)
    full = lambda p, b: (0, 0)
    out = pl.pallas_call(
        body,
        grid=(2, NB),
        in_specs=[
            pl.BlockSpec((BN, N), lambda p, b: (b, 0)),
            pl.BlockSpec((N, F), full),
            pl.BlockSpec((F, H), full),
            pl.BlockSpec((F, H), full),
            pl.BlockSpec((1, H), full),
            pl.BlockSpec((H, K), full),
            pl.BlockSpec((1, K), full),
            pl.BlockSpec((H, H), full),
            pl.BlockSpec((H, H), full),
            pl.BlockSpec((1, H), full),
            pl.BlockSpec((H, 1), full),
            pl.BlockSpec((1, 1), full),
        ],
        out_specs=pl.BlockSpec((K, 1), full),
        out_shape=jax.ShapeDtypeStruct((K, 1), jnp.float32),
        scratch_shapes=[
            pltpu.VMEM((N, H), jnp.float32),
            pltpu.VMEM((N, N), jnp.bfloat16),
            pltpu.VMEM((N, K), jnp.bfloat16),
            pltpu.VMEM((K, H), jnp.float32),
            pltpu.VMEM((K, K), jnp.float32),
        ],
    )(a, x, W1a, W1b, b1.reshape(1, H), Wp, bp.reshape(1, K),
      W2a, W2b, b2.reshape(1, H), Wd, bd.reshape(1, 1))
    return out
